# Initial kernel scaffold; baseline (speedup 1.0000x reference)
import jax
import jax.numpy as jnp
from jax.experimental import pallas as pl


def kernel(x, Wg, Wc1, bc1, Wc2, bc2, gamma, beta, fc1s, b1s, fc2s, b2s):
    raise NotImplementedError("write your pallas kernel here")


# speedup vs baseline: 1.8623x; 1.8623x over previous
import functools

import jax
import jax.numpy as jnp
from jax import lax
from jax.experimental import pallas as pl
from jax.experimental.pallas import tpu as pltpu
from jax.experimental.pallas import tpu_sc as plsc


def _gelu(h):
    return jax.nn.gelu(h, approximate=True)


def _k1_body(x_ref, wg_ref, wc1_ref, bc1_ref, wc2_ref, bc2_ref,
             scores_ref, logits_ref, spsum_ref, spacc):
    i = pl.program_id(0)
    xb = x_ref[...]
    g = lax.dot_general(xb, wg_ref[...], (((1,), (1,)), ((), ())),
                        preferred_element_type=jnp.float32)
    scores_ref[...] = (jnp.tanh(g) + 1.0) * 0.5
    h = lax.dot_general(xb, wc1_ref[...], (((1,), (0,)), ((), ())),
                        preferred_element_type=jnp.float32) + bc1_ref[...]
    h = _gelu(h)
    l = lax.dot_general(h, wc2_ref[...], (((1,), (1,)), ((), ())),
                        preferred_element_type=jnp.float32) + bc2_ref[...]
    logits_ref[...] = l
    sp = jnp.sum(jnp.maximum(l, 0.0) + jnp.log1p(jnp.exp(-jnp.abs(l))))

    @pl.when(i == 0)
    def _():
        spacc[0] = 0.0

    spacc[0] += sp
    spsum_ref[...] = jnp.reshape(spacc[0], (1, 1))


def _gate_cap(xf, Wg, Wc1, bc1, Wc2, bc2, rb):
    bs, d = xf.shape
    e = Wg.shape[0]
    nb = bs // rb
    return pl.pallas_call(
        _k1_body,
        grid=(nb,),
        in_specs=[
            pl.BlockSpec((rb, d), lambda i: (i, 0)),
            pl.BlockSpec((e, d), lambda i: (0, 0)),
            pl.BlockSpec((d, d), lambda i: (0, 0)),
            pl.BlockSpec((1, d), lambda i: (0, 0)),
            pl.BlockSpec((e, d), lambda i: (0, 0)),
            pl.BlockSpec((1, e), lambda i: (0, 0)),
        ],
        out_specs=[
            pl.BlockSpec((rb, e), lambda i: (i, 0)),
            pl.BlockSpec((rb, e), lambda i: (i, 0)),
            pl.BlockSpec((1, 1), lambda i: (0, 0)),
        ],
        out_shape=[
            jax.ShapeDtypeStruct((bs, e), jnp.float32),
            jax.ShapeDtypeStruct((bs, e), jnp.float32),
            jax.ShapeDtypeStruct((1, 1), jnp.float32),
        ],
        scratch_shapes=[pltpu.SMEM((1,), jnp.float32)],
    )(xf, Wg, Wc1, bc1.reshape(1, d), Wc2, bc2.reshape(1, e))


def _sc_gather(table, idx):
    n, d = table.shape
    b = idx.shape[0]
    nw = 32
    b_per_w = b // nw
    chunk = 64
    nch = b_per_w // chunk
    mesh = plsc.VectorSubcoreMesh(core_axis_name="c", subcore_axis_name="s")

    @functools.partial(
        pl.kernel,
        mesh=mesh,
        out_type=jax.ShapeDtypeStruct((b, d), jnp.float32),
        scratch_types=[
            pltpu.VMEM((chunk,), jnp.int32),
            pltpu.VMEM((chunk, d), jnp.float32),
            pltpu.SemaphoreType.DMA,
        ],
    )
    def gather_k(table_hbm, idx_hbm, out_hbm, idx_v, rows_v, sem):
        wid = lax.axis_index("s") * 2 + lax.axis_index("c")
        base = wid * b_per_w
        for c in range(nch):
            off = base + c * chunk
            pltpu.sync_copy(idx_hbm.at[pl.ds(off, chunk)], idx_v)
            pltpu.async_copy(table_hbm.at[idx_v], rows_v, sem).wait()
            pltpu.sync_copy(rows_v, out_hbm.at[pl.ds(off, chunk)])

    return gather_k(table, idx)


def _k2_body(xg_ref, w1_ref, b1_ref, w2_ref, b2_ref, vals_ref, gamma_ref,
             beta_ref, y_ref, xn_ref, *, nd):
    d = pl.program_id(1)

    @pl.when(d == 0)
    def _():
        xb = xg_ref[0]
        m = jnp.mean(xb, axis=-1, keepdims=True)
        v = jnp.mean((xb - m) ** 2, axis=-1, keepdims=True)
        xn_ref[...] = ((xb - m) * lax.rsqrt(v + 1e-5)) * gamma_ref[...] + beta_ref[...]

    h = lax.dot_general(xn_ref[...], w1_ref[0], (((1,), (1,)), ((), ())),
                        preferred_element_type=jnp.float32)
    h = _gelu(h + b1_ref[0, 0])
    contrib = lax.dot_general(h, w2_ref[0], (((1,), (1,)), ((), ())),
                              preferred_element_type=jnp.float32)

    @pl.when(d == 0)
    def _():
        y_ref[0] = contrib

    @pl.when(jnp.logical_and(d > 0, d < nd - 1))
    def _():
        y_ref[0] = y_ref[0] + contrib

    @pl.when(d == nd - 1)
    def _():
        y_ref[0] = (y_ref[0] + contrib + b2_ref[0, 0]) * vals_ref[0, 0][:, None]


def _expert_mlp(xg, vals, gamma, beta, fc1s, b1s, fc2s, b2s, dblk):
    e, ff, d = fc1s.shape
    k = xg.shape[0] // e
    nd = ff // dblk
    xg3 = xg.reshape(e, k, d)
    return pl.pallas_call(
        functools.partial(_k2_body, nd=nd),
        grid=(e, nd),
        in_specs=[
            pl.BlockSpec((1, k, d), lambda i, j: (i, 0, 0)),
            pl.BlockSpec((1, dblk, d), lambda i, j: (i, j, 0)),
            pl.BlockSpec((1, 1, 1, dblk), lambda i, j: (i, j, 0, 0)),
            pl.BlockSpec((1, d, dblk), lambda i, j: (i, 0, j)),
            pl.BlockSpec((1, 1, d), lambda i, j: (i, 0, 0)),
            pl.BlockSpec((1, 1, k), lambda i, j: (i, 0, 0)),
            pl.BlockSpec((1, d), lambda i, j: (0, 0)),
            pl.BlockSpec((1, d), lambda i, j: (0, 0)),
        ],
        out_specs=pl.BlockSpec((1, k, d), lambda i, j: (i, 0, 0)),
        out_shape=jax.ShapeDtypeStruct((e, k, d), jnp.float32),
        scratch_shapes=[pltpu.VMEM((k, d), jnp.float32)],
    )(xg3, fc1s, b1s.reshape(e, nd, 1, dblk), fc2s, b2s.reshape(e, 1, d),
      vals.reshape(e, 1, k), gamma.reshape(1, d), beta.reshape(1, d))


def _k3_body(idx_ref, xf_ref, y_ref, lg_ref, out_ref, selsum_ref, acc, selacc,
             *, e, k, wb):
    i = pl.program_id(0)

    @pl.when(i == 0)
    def _():
        acc[...] = jnp.zeros_like(acc)
        selacc[0] = 0.0

    @pl.when(i < e)
    def _():
        oh = (lax.broadcasted_iota(jnp.int32, (1, e), 1) == i).astype(jnp.float32)

        def body(j, svec):
            tok = idx_ref[i * k + j]
            acc[pl.ds(tok, 1)] = acc[pl.ds(tok, 1)] + y_ref[pl.ds(j, 1)]
            return svec + lg_ref[pl.ds(tok, 1), :]

        svec = lax.fori_loop(0, k, body, jnp.zeros((1, e), jnp.float32))
        selacc[0] += jnp.sum(svec * oh)
        selsum_ref[...] = jnp.reshape(selacc[0], (1, 1))

    @pl.when(i >= e)
    def _():
        out_ref[...] = xf_ref[...] + acc[pl.ds((i - e) * wb, wb)]


def _scatter_add(flat_idx, xf3, y3, logits, wb):
    bs = xf3.shape[0]
    e = logits.shape[1]
    k = bs // e
    nwb = bs // wb
    grid_spec = pltpu.PrefetchScalarGridSpec(
        num_scalar_prefetch=1,
        grid=(e + nwb,),
        in_specs=[
            pl.BlockSpec((wb, 8, 128), lambda i, idx: (jnp.maximum(i - e, 0), 0, 0)),
            pl.BlockSpec((k, 8, 128), lambda i, idx: (jnp.minimum(i, e - 1), 0, 0)),
            pl.BlockSpec((bs, e), lambda i, idx: (0, 0)),
        ],
        out_specs=[
            pl.BlockSpec((wb, 8, 128), lambda i, idx: (jnp.maximum(i - e, 0), 0, 0)),
            pl.BlockSpec((1, 1), lambda i, idx: (0, 0)),
        ],
        scratch_shapes=[
            pltpu.VMEM((bs, 8, 128), jnp.float32),
            pltpu.SMEM((1,), jnp.float32),
        ],
    )
    return pl.pallas_call(
        functools.partial(_k3_body, e=e, k=k, wb=wb),
        grid_spec=grid_spec,
        out_shape=[
            jax.ShapeDtypeStruct((bs, 8, 128), jnp.float32),
            jax.ShapeDtypeStruct((1, 1), jnp.float32),
        ],
    )(flat_idx, xf3, y3, logits)


def kernel(x, Wg, Wc1, bc1, Wc2, bc2, gamma, beta, fc1s, b1s, fc2s, b2s):
    og_shape = x.shape
    d = og_shape[-1]
    xf = x.reshape(-1, d)
    bs = xf.shape[0]
    e = Wg.shape[0]
    k = bs // e

    scores, logits, spsum = _gate_cap(xf, Wg, Wc1, bc1, Wc2, bc2, rb=512)
    vals, idx = lax.top_k(scores.T, k)
    flat_idx = idx.reshape(bs).astype(jnp.int32)

    xg = _sc_gather(xf, flat_idx)
    y = _expert_mlp(xg, vals, gamma, beta, fc1s, b1s, fc2s, b2s, dblk=1024)

    xf3 = xf.reshape(bs, 8, d // 8)
    y3 = y.reshape(bs, 8, d // 8)
    out3, selsum = _scatter_add(flat_idx, xf3, y3, logits, wb=512)

    cap_loss = (spsum[0, 0] - selsum[0, 0]) / (bs * e)
    return out3.reshape(og_shape), cap_loss


# speedup vs baseline: 1.9127x; 1.0271x over previous
import functools

import jax
import jax.numpy as jnp
from jax import lax
from jax.experimental import pallas as pl
from jax.experimental.pallas import tpu as pltpu
from jax.experimental.pallas import tpu_sc as plsc


def _gelu(h):
    return jax.nn.gelu(h, approximate=True)


def _k1_body(x_ref, wg_ref, wc1_ref, bc1_ref, wc2_ref, bc2_ref, gamma_ref,
             beta_ref, scores_ref, logits_ref, spsum_ref, xn_ref, spacc):
    i = pl.program_id(0)
    xb = x_ref[...]
    g = lax.dot_general(xb, wg_ref[...], (((1,), (1,)), ((), ())),
                        preferred_element_type=jnp.float32)
    scores_ref[...] = (jnp.tanh(g) + 1.0) * 0.5
    m = jnp.mean(xb, axis=-1, keepdims=True)
    v = jnp.mean((xb - m) ** 2, axis=-1, keepdims=True)
    xn_ref[...] = ((xb - m) * lax.rsqrt(v + 1e-5)) * gamma_ref[...] + beta_ref[...]
    h = lax.dot_general(xb.astype(jnp.bfloat16),
                        wc1_ref[...].astype(jnp.bfloat16),
                        (((1,), (0,)), ((), ())),
                        preferred_element_type=jnp.float32) + bc1_ref[...]
    h = _gelu(h)
    l = lax.dot_general(h, wc2_ref[...], (((1,), (1,)), ((), ())),
                        preferred_element_type=jnp.float32) + bc2_ref[...]
    logits_ref[...] = l
    sp = jnp.sum(jnp.maximum(l, 0.0) + jnp.log1p(jnp.exp(-jnp.abs(l))))

    @pl.when(i == 0)
    def _():
        spacc[0] = 0.0

    spacc[0] += sp
    spsum_ref[...] = jnp.reshape(spacc[0], (1, 1))


def _gate_cap(xf, Wg, Wc1, bc1, Wc2, bc2, gamma, beta, rb):
    bs, d = xf.shape
    e = Wg.shape[0]
    nb = bs // rb
    return pl.pallas_call(
        _k1_body,
        grid=(nb,),
        in_specs=[
            pl.BlockSpec((rb, d), lambda i: (i, 0)),
            pl.BlockSpec((e, d), lambda i: (0, 0)),
            pl.BlockSpec((d, d), lambda i: (0, 0)),
            pl.BlockSpec((1, d), lambda i: (0, 0)),
            pl.BlockSpec((e, d), lambda i: (0, 0)),
            pl.BlockSpec((1, e), lambda i: (0, 0)),
            pl.BlockSpec((1, d), lambda i: (0, 0)),
            pl.BlockSpec((1, d), lambda i: (0, 0)),
        ],
        out_specs=[
            pl.BlockSpec((rb, e), lambda i: (i, 0)),
            pl.BlockSpec((rb, e), lambda i: (i, 0)),
            pl.BlockSpec((1, 1), lambda i: (0, 0)),
            pl.BlockSpec((rb, d), lambda i: (i, 0)),
        ],
        out_shape=[
            jax.ShapeDtypeStruct((bs, e), jnp.float32),
            jax.ShapeDtypeStruct((bs, e), jnp.float32),
            jax.ShapeDtypeStruct((1, 1), jnp.float32),
            jax.ShapeDtypeStruct((bs, d), jnp.float32),
        ],
        scratch_shapes=[pltpu.SMEM((1,), jnp.float32)],
    )(xf, Wg, Wc1, bc1.reshape(1, d), Wc2,
      bc2.reshape(1, e), gamma.reshape(1, d), beta.reshape(1, d))


def _sc_gather(table, idx):
    n, d = table.shape
    b = idx.shape[0]
    nw = 32
    b_per_w = b // nw
    chunk = 64
    nch = b_per_w // chunk
    mesh = plsc.VectorSubcoreMesh(core_axis_name="c", subcore_axis_name="s")

    @functools.partial(
        pl.kernel,
        mesh=mesh,
        out_type=jax.ShapeDtypeStruct((b, d), jnp.float32),
        scratch_types=[
            pltpu.VMEM((chunk,), jnp.int32),
            pltpu.VMEM((chunk, d), jnp.float32),
            pltpu.SemaphoreType.DMA,
        ],
    )
    def gather_k(table_hbm, idx_hbm, out_hbm, idx_v, rows_v, sem):
        wid = lax.axis_index("s") * 2 + lax.axis_index("c")
        base = wid * b_per_w
        for c in range(nch):
            off = base + c * chunk
            pltpu.sync_copy(idx_hbm.at[pl.ds(off, chunk)], idx_v)
            pltpu.async_copy(table_hbm.at[idx_v], rows_v, sem).wait()
            pltpu.sync_copy(rows_v, out_hbm.at[pl.ds(off, chunk)])

    return gather_k(table, idx)


def _k2_body(xg_ref, w1_ref, b1_ref, w2_ref, b2_ref, vals_ref, y_ref, xbf_ref,
             *, nd):
    d = pl.program_id(1)

    @pl.when(d == 0)
    def _():
        xbf_ref[...] = xg_ref[0].astype(jnp.bfloat16)

    h = lax.dot_general(xbf_ref[...], w1_ref[0].astype(jnp.bfloat16),
                        (((1,), (1,)), ((), ())),
                        preferred_element_type=jnp.float32)
    hb = (h + b1_ref[0, 0]).astype(jnp.bfloat16)
    contrib = lax.dot_general(_gelu(hb),
                              w2_ref[0].astype(jnp.bfloat16),
                              (((1,), (1,)), ((), ())),
                              preferred_element_type=jnp.float32)

    @pl.when(d == 0)
    def _():
        y_ref[0] = contrib

    @pl.when(jnp.logical_and(d > 0, d < nd - 1))
    def _():
        y_ref[0] = y_ref[0] + contrib

    @pl.when(d == nd - 1)
    def _():
        y_ref[0] = (y_ref[0] + contrib + b2_ref[0, 0]) * vals_ref[0, 0][:, None]


def _expert_mlp(xg, vals, fc1s, b1s, fc2s, b2s, dblk):
    e, ff, d = fc1s.shape
    k = xg.shape[0] // e
    nd = ff // dblk
    xg3 = xg.reshape(e, k, d)
    return pl.pallas_call(
        functools.partial(_k2_body, nd=nd),
        grid=(e, nd),
        in_specs=[
            pl.BlockSpec((1, k, d), lambda i, j: (i, 0, 0)),
            pl.BlockSpec((1, dblk, d), lambda i, j: (i, j, 0)),
            pl.BlockSpec((1, 1, 1, dblk), lambda i, j: (i, j, 0, 0)),
            pl.BlockSpec((1, d, dblk), lambda i, j: (i, 0, j)),
            pl.BlockSpec((1, 1, d), lambda i, j: (i, 0, 0)),
            pl.BlockSpec((1, 1, k), lambda i, j: (i, 0, 0)),
        ],
        out_specs=pl.BlockSpec((1, k, d), lambda i, j: (i, 0, 0)),
        out_shape=jax.ShapeDtypeStruct((e, k, d), jnp.float32),
        scratch_shapes=[pltpu.VMEM((k, d), jnp.bfloat16)],
    )(xg3, fc1s, b1s.reshape(e, nd, 1, dblk), fc2s, b2s.reshape(e, 1, d),
      vals.reshape(e, 1, k))


def _k3_body(idx_ref, xf_ref, y_ref, lg_ref, out_ref, selsum_ref, acc, selacc,
             *, e, k, wb):
    i = pl.program_id(0)

    @pl.when(i == 0)
    def _():
        acc[...] = jnp.zeros_like(acc)
        selacc[0] = 0.0

    @pl.when(i < e)
    def _():
        oh = (lax.broadcasted_iota(jnp.int32, (1, e), 1) == i).astype(jnp.float32)

        def body(j, svec):
            tok = idx_ref[i * k + j]
            acc[pl.ds(tok, 1)] = acc[pl.ds(tok, 1)] + y_ref[pl.ds(j, 1)]
            return svec + lg_ref[pl.ds(tok, 1), :]

        svec = lax.fori_loop(0, k, body, jnp.zeros((1, e), jnp.float32))
        selacc[0] += jnp.sum(svec * oh)
        selsum_ref[...] = jnp.reshape(selacc[0], (1, 1))

    @pl.when(i >= e)
    def _():
        out_ref[...] = xf_ref[...] + acc[pl.ds((i - e) * wb, wb)]


def _scatter_add(flat_idx, xf3, y3, logits, wb):
    bs = xf3.shape[0]
    e = logits.shape[1]
    k = bs // e
    nwb = bs // wb
    grid_spec = pltpu.PrefetchScalarGridSpec(
        num_scalar_prefetch=1,
        grid=(e + nwb,),
        in_specs=[
            pl.BlockSpec((wb, 8, 128), lambda i, idx: (jnp.maximum(i - e, 0), 0, 0)),
            pl.BlockSpec((k, 8, 128), lambda i, idx: (jnp.minimum(i, e - 1), 0, 0)),
            pl.BlockSpec((bs, e), lambda i, idx: (0, 0)),
        ],
        out_specs=[
            pl.BlockSpec((wb, 8, 128), lambda i, idx: (jnp.maximum(i - e, 0), 0, 0)),
            pl.BlockSpec((1, 1), lambda i, idx: (0, 0)),
        ],
        scratch_shapes=[
            pltpu.VMEM((bs, 8, 128), jnp.float32),
            pltpu.SMEM((1,), jnp.float32),
        ],
    )
    return pl.pallas_call(
        functools.partial(_k3_body, e=e, k=k, wb=wb),
        grid_spec=grid_spec,
        out_shape=[
            jax.ShapeDtypeStruct((bs, 8, 128), jnp.float32),
            jax.ShapeDtypeStruct((1, 1), jnp.float32),
        ],
    )(flat_idx, xf3, y3, logits)


def kernel(x, Wg, Wc1, bc1, Wc2, bc2, gamma, beta, fc1s, b1s, fc2s, b2s):
    og_shape = x.shape
    d = og_shape[-1]
    xf = x.reshape(-1, d)
    bs = xf.shape[0]
    e = Wg.shape[0]
    k = bs // e

    scores, logits, spsum, xnorm = _gate_cap(xf, Wg, Wc1, bc1, Wc2, bc2,
                                             gamma, beta, rb=512)
    vals, idx = lax.top_k(scores.T, k)
    flat_idx = idx.reshape(bs).astype(jnp.int32)

    xg = _sc_gather(xnorm, flat_idx)
    y = _expert_mlp(xg, vals, fc1s, b1s, fc2s, b2s, dblk=1024)

    xf3 = xf.reshape(bs, 8, d // 8)
    y3 = y.reshape(bs, 8, d // 8)
    out3, selsum = _scatter_add(flat_idx, xf3, y3, logits, wb=512)

    cap_loss = (spsum[0, 0] - selsum[0, 0]) / (bs * e)
    return out3.reshape(og_shape), cap_loss
